# baseline (speedup 1.0000x reference)
import functools

import jax
import jax.numpy as jnp
from jax import lax
from jax.experimental import pallas as pl
from jax.experimental.pallas import tpu as pltpu
from jax.experimental.pallas import tpu_sc as plsc

B, N, M, CIN, COUT = 4, 1024, 4096, 256, 128
CIN2 = COUT
NW = 32
MPW = B * M // NW
NG = MPW // 16
BN_EPS = 1e-5
_HIGH = jax.lax.Precision.HIGHEST


def _h_body(x1_ref, wt_ref, g_ref, b_ref, out_ref):
    wt = wt_ref[...]
    ys = [
        lax.dot_general(x1_ref[b], wt, (((0,), (0,)), ((), ())),
                        preferred_element_type=jnp.float32, precision=_HIGH)
        for b in range(B)
    ]
    tot = ys[0] + ys[1] + ys[2] + ys[3]
    mean = jnp.sum(tot, axis=0, keepdims=True) * (1.0 / (B * N))
    sq = sum(jnp.sum(y * y, axis=0, keepdims=True) for y in ys)
    var = sq * (1.0 / (B * N)) - mean * mean
    scale = g_ref[...] / jnp.sqrt(var + BN_EPS)
    shift = b_ref[...] - mean * scale
    for b in range(B):
        out_ref[b] = jnp.maximum(ys[b] * scale + shift, 0.0)


def _h_call(x1, wup_t, g1, b1):
    return pl.pallas_call(
        _h_body,
        out_shape=jax.ShapeDtypeStruct((B, N, COUT), jnp.float32),
    )(x1, wup_t, g1, b1)


def _l_body(x2_ref, wt_ref, g_ref, b_ref, out_ref, cov_ref, mu_ref):
    i = pl.program_id(0)

    @pl.when(i == 0)
    def _init():
        cov_ref[...] = jnp.zeros((CIN2, CIN2), jnp.float32)
        mu_ref[...] = jnp.zeros((CIN2, 1), jnp.float32)

    xb = x2_ref[0]

    @pl.when(i < B)
    def _accum():
        cov_ref[...] += lax.dot_general(
            xb, xb, (((1,), (1,)), ((), ())),
            preferred_element_type=jnp.float32, precision=_HIGH)
        mu_ref[...] += jnp.sum(xb, axis=1, keepdims=True)
        out_ref[...] = jnp.zeros_like(out_ref)

    @pl.when(i >= B)
    def _emit():
        wt = wt_ref[...]
        inv_l = 1.0 / (B * M)
        mean = lax.dot_general(mu_ref[...], wt, (((0,), (0,)), ((), ())),
                               preferred_element_type=jnp.float32,
                               precision=_HIGH) * inv_l
        t = lax.dot_general(cov_ref[...], wt, (((1,), (0,)), ((), ())),
                            preferred_element_type=jnp.float32,
                            precision=_HIGH)
        e2 = jnp.sum(t * wt, axis=0, keepdims=True) * inv_l
        var = e2 - mean * mean
        scale = g_ref[...] / jnp.sqrt(var + BN_EPS)
        shift = b_ref[...] - mean * scale
        wf = wt * scale
        y = lax.dot_general(xb, wf, (((0,), (0,)), ((), ())),
                            preferred_element_type=jnp.float32,
                            precision=_HIGH)
        out_ref[...] = jnp.maximum(y + shift, 0.0)[None]


def _l_call(x2, wlat_t, g2, b2):
    return pl.pallas_call(
        _l_body,
        grid=(2 * B,),
        in_specs=[
            pl.BlockSpec((1, CIN2, M), lambda i: (lax.rem(i, B), 0, 0)),
            pl.BlockSpec((CIN2, COUT), lambda i: (0, 0)),
            pl.BlockSpec((1, COUT), lambda i: (0, 0)),
            pl.BlockSpec((1, COUT), lambda i: (0, 0)),
        ],
        out_specs=pl.BlockSpec(
            (1, M, COUT), lambda i: (jnp.maximum(i - B, 0), 0, 0)),
        out_shape=jax.ShapeDtypeStruct((B, M, COUT), jnp.float32),
        scratch_shapes=[
            pltpu.VMEM((CIN2, CIN2), jnp.float32),
            pltpu.VMEM((CIN2, 1), jnp.float32),
        ],
    )(x2, wlat_t, g2, b2)


def _bf16_round(x):
    u = plsc.bitcast(x, jnp.uint32)
    r = (u + jnp.uint32(0x7FFF) + ((u >> jnp.uint32(16)) & jnp.uint32(1)))
    r = r & jnp.uint32(0xFFFF0000)
    return plsc.bitcast(r, jnp.float32)


def _sc_body(p1x, p1y, p1z, p2x, p2y, p2z, h_hbm, l_hbm, out_hbm,
             cx, cy, cz, cn, cbx, cby, cbz, cbn, fxr, fyr, fzr,
             wb0, wb1, wb2, ib0, ib1, ib2,
             rows0, rows1, rows2, acc, sem0, sem1, sem2):
    nc = 2
    wid = lax.axis_index("s") * nc + lax.axis_index("c")
    b = wid // (NW // B)
    mbase = b * M + (wid % (NW // B)) * MPW
    grow0 = wid * MPW

    pltpu.sync_copy(p1x.at[pl.ds(b * N, N)], cx)
    pltpu.sync_copy(p1y.at[pl.ds(b * N, N)], cy)
    pltpu.sync_copy(p1z.at[pl.ds(b * N, N)], cz)
    pltpu.sync_copy(p2x.at[pl.ds(mbase, MPW)], fxr)
    pltpu.sync_copy(p2y.at[pl.ds(mbase, MPW)], fyr)
    pltpu.sync_copy(p2z.at[pl.ds(mbase, MPW)], fzr)

    def _cn_body(i, _):
        sl = pl.ds(i * 16, 16)
        vx = cx[sl]
        vy = cy[sl]
        vz = cz[sl]
        cn[sl] = vx * vx + vy * vy + vz * vz
        cx[sl] = _bf16_round(vx) * 2.0
        cy[sl] = _bf16_round(vy) * 2.0
        cz[sl] = _bf16_round(vz) * 2.0
        return 0

    lax.fori_loop(0, N // 16, _cn_body, 0)

    def _bcast(n, _):
        nv = jnp.broadcast_to(n, (16,)).astype(jnp.int32)
        sl = pl.ds(n * 16, 16)
        cbx[sl] = plsc.load_gather(cx, [nv])
        cby[sl] = plsc.load_gather(cy, [nv])
        cbz[sl] = plsc.load_gather(cz, [nv])
        cbn[sl] = plsc.load_gather(cn, [nv])
        return 0

    lax.fori_loop(0, N, _bcast, 0)

    big = jnp.full((16,), 3.0e38, jnp.float32)
    zero_i = jnp.zeros((16,), jnp.int32)

    def _group(g, _):
        sl = pl.ds(g * 16, 16)
        fx = fxr[sl]
        fy = fyr[sl]
        fz = fzr[sl]
        fn = fx * fx + fy * fy + fz * fz
        fx = _bf16_round(fx)
        fy = _bf16_round(fy)
        fz = _bf16_round(fz)

        kbig = jnp.full((16,), 0xFFFFFFFF, jnp.uint32)
        hi = jnp.full((16,), 0x80000000, jnp.uint32)
        msk = jnp.full((16,), 0xFFFFFC00, jnp.uint32)

        def _scan(n, carry):
            k1, k2, k3 = carry
            nv = jnp.broadcast_to(n, (16,)).astype(jnp.int32)
            nsl = pl.ds(n * 16, 16)
            cxv = cbx[nsl]
            cyv = cby[nsl]
            czv = cbz[nsl]
            cnv = cbn[nsl]
            dot2 = fx * cxv + fy * cyv + fz * czv
            d = (fn + cnv) - dot2
            di = plsc.bitcast(d, jnp.int32)
            sgn = plsc.bitcast(di >> 31, jnp.uint32)
            key = plsc.bitcast(di, jnp.uint32) ^ (sgn | hi)
            key = (key & msk) | plsc.bitcast(nv, jnp.uint32)
            a = jnp.minimum(key, k1)
            bq = jnp.maximum(key, k1)
            c = jnp.minimum(bq, k2)
            e = jnp.maximum(bq, k2)
            return a, c, jnp.minimum(e, k3)

        k1, k2, k3 = lax.fori_loop(
            0, N, _scan, (kbig, kbig, kbig), unroll=4)

        base = jnp.broadcast_to(b * N, (16,)).astype(jnp.int32)
        idxm = jnp.full((16,), 0x3FF, jnp.uint32)
        rs = []
        for t, kk in enumerate((k1, k2, k3)):
            ik = plsc.bitcast(kk & idxm, jnp.int32)
            cxv = plsc.load_gather(cx, [ik])
            cyv = plsc.load_gather(cy, [ik])
            czv = plsc.load_gather(cz, [ik])
            cnv = plsc.load_gather(cn, [ik])
            dk = (fn + cnv) - (fx * cxv + fy * cyv + fz * czv)
            rs.append(1.0 / (dk + 1e-8))
            [ib0, ib1, ib2][t][sl] = ik + base
        s = 1.0 / (rs[0] + rs[1] + rs[2])
        wb0[sl] = rs[0] * s
        wb1[sl] = rs[1] * s
        wb2[sl] = rs[2] * s
        return 0

    lax.fori_loop(0, NG, _group, 0)

    rows = [rows0, rows1, rows2]
    sems = [sem0, sem1, sem2]
    ibufs = [ib0, ib1, ib2]
    chunk = 64
    for c in range(MPW // chunk):
        cps = []
        for k in range(3):
            cp = pltpu.make_async_copy(
                h_hbm.at[ibufs[k].at[pl.ds(c * chunk, chunk)]], rows[k],
                sems[k])
            cp.start()
            cps.append(cp)
        pltpu.sync_copy(
            l_hbm.at[pl.ds((grow0 + c * chunk) * COUT, chunk * COUT)], acc)
        for cp in cps:
            cp.wait()

        def _mrow(m, _):
            mv = jnp.broadcast_to(c * chunk + m, (16,)).astype(jnp.int32)
            w0 = plsc.load_gather(wb0, [mv])
            w1 = plsc.load_gather(wb1, [mv])
            w2 = plsc.load_gather(wb2, [mv])
            for q in range(8):
                csl = pl.ds(q * 16, 16)
                g0 = rows0[m, csl]
                g1 = rows1[m, csl]
                g2 = rows2[m, csl]
                sl = pl.ds(m * COUT + q * 16, 16)
                acc[sl] = acc[sl] + w0 * g0 + w1 * g1 + w2 * g2
            return 0

        lax.fori_loop(0, chunk, _mrow, 0)
        pltpu.sync_copy(
            acc, out_hbm.at[pl.ds((grow0 + c * chunk) * COUT, chunk * COUT)])


def _sc_call(p1f, p2f, h_rows, l_flat):
    mesh = plsc.VectorSubcoreMesh(core_axis_name="c", subcore_axis_name="s")
    fn = pl.kernel(
        _sc_body,
        out_type=jax.ShapeDtypeStruct((B * M * COUT,), jnp.float32),
        mesh=mesh,
        compiler_params=pltpu.CompilerParams(
            use_tc_tiling_on_sc=False, needs_layout_passes=False),
        scratch_types=[
            pltpu.VMEM((N,), jnp.float32),
            pltpu.VMEM((N,), jnp.float32),
            pltpu.VMEM((N,), jnp.float32),
            pltpu.VMEM((N,), jnp.float32),
            pltpu.VMEM((N * 16,), jnp.float32),
            pltpu.VMEM((N * 16,), jnp.float32),
            pltpu.VMEM((N * 16,), jnp.float32),
            pltpu.VMEM((N * 16,), jnp.float32),
            pltpu.VMEM((MPW,), jnp.float32),
            pltpu.VMEM((MPW,), jnp.float32),
            pltpu.VMEM((MPW,), jnp.float32),
            pltpu.VMEM((MPW,), jnp.float32),
            pltpu.VMEM((MPW,), jnp.float32),
            pltpu.VMEM((MPW,), jnp.float32),
            pltpu.VMEM((MPW,), jnp.int32),
            pltpu.VMEM((MPW,), jnp.int32),
            pltpu.VMEM((MPW,), jnp.int32),
            pltpu.VMEM((64, COUT), jnp.float32),
            pltpu.VMEM((64, COUT), jnp.float32),
            pltpu.VMEM((64, COUT), jnp.float32),
            pltpu.VMEM((64 * COUT,), jnp.float32),
            pltpu.SemaphoreType.DMA,
            pltpu.SemaphoreType.DMA,
            pltpu.SemaphoreType.DMA,
        ],
    )
    return fn(p1f[0], p1f[1], p1f[2], p2f[0], p2f[1], p2f[2], h_rows, l_flat)


@jax.jit
def kernel(x1, p1, x2, p2, W_up, gamma1, beta1, W_lat, gamma2, beta2):
    p1f = jnp.transpose(p1, (2, 0, 1)).reshape(3, B * N)
    p2f = jnp.transpose(p2, (2, 0, 1)).reshape(3, B * M)
    h = _h_call(x1, W_up.T, gamma1.reshape(1, COUT), beta1.reshape(1, COUT))
    l = _l_call(x2, W_lat.T, gamma2.reshape(1, COUT), beta2.reshape(1, COUT))
    out_flat = _sc_call(p1f, p2f, h.reshape(B * N, COUT),
                        l.reshape(B * M * COUT))
    out = jnp.transpose(out_flat.reshape(B, M, COUT), (0, 2, 1))
    return (out, p2)


# speedup vs baseline: 1.0711x; 1.0711x over previous
import functools

import jax
import jax.numpy as jnp
from jax import lax
from jax.experimental import pallas as pl
from jax.experimental.pallas import tpu as pltpu
from jax.experimental.pallas import tpu_sc as plsc

B, N, M, CIN, COUT = 4, 1024, 4096, 256, 128
CIN2 = COUT
NW = 32
MPW = B * M // NW
NG = MPW // 16
BN_EPS = 1e-5
_HIGH = jax.lax.Precision.HIGHEST


def _h_body(x1_ref, wt_ref, g_ref, b_ref, out_ref):
    wt = wt_ref[...]
    ys = [
        lax.dot_general(x1_ref[b], wt, (((0,), (0,)), ((), ())),
                        preferred_element_type=jnp.float32, precision=_HIGH)
        for b in range(B)
    ]
    tot = ys[0] + ys[1] + ys[2] + ys[3]
    mean = jnp.sum(tot, axis=0, keepdims=True) * (1.0 / (B * N))
    sq = sum(jnp.sum(y * y, axis=0, keepdims=True) for y in ys)
    var = sq * (1.0 / (B * N)) - mean * mean
    scale = g_ref[...] / jnp.sqrt(var + BN_EPS)
    shift = b_ref[...] - mean * scale
    for b in range(B):
        out_ref[b] = jnp.maximum(ys[b] * scale + shift, 0.0)


def _h_call(x1, wup_t, g1, b1):
    return pl.pallas_call(
        _h_body,
        out_shape=jax.ShapeDtypeStruct((B, N, COUT), jnp.float32),
    )(x1, wup_t, g1, b1)


def _l_body(x2_ref, wt_ref, g_ref, b_ref, out_ref, cov_ref, mu_ref):
    i = pl.program_id(0)

    @pl.when(i == 0)
    def _init():
        cov_ref[...] = jnp.zeros((CIN2, CIN2), jnp.float32)
        mu_ref[...] = jnp.zeros((CIN2, 1), jnp.float32)

    xb = x2_ref[0]

    @pl.when(i < B)
    def _accum():
        cov_ref[...] += lax.dot_general(
            xb, xb, (((1,), (1,)), ((), ())),
            preferred_element_type=jnp.float32, precision=_HIGH)
        mu_ref[...] += jnp.sum(xb, axis=1, keepdims=True)
        out_ref[...] = jnp.zeros_like(out_ref)

    @pl.when(i >= B)
    def _emit():
        wt = wt_ref[...]
        inv_l = 1.0 / (B * M)
        mean = lax.dot_general(mu_ref[...], wt, (((0,), (0,)), ((), ())),
                               preferred_element_type=jnp.float32,
                               precision=_HIGH) * inv_l
        t = lax.dot_general(cov_ref[...], wt, (((1,), (0,)), ((), ())),
                            preferred_element_type=jnp.float32,
                            precision=_HIGH)
        e2 = jnp.sum(t * wt, axis=0, keepdims=True) * inv_l
        var = e2 - mean * mean
        scale = g_ref[...] / jnp.sqrt(var + BN_EPS)
        shift = b_ref[...] - mean * scale
        wf = wt * scale
        y = lax.dot_general(xb, wf, (((0,), (0,)), ((), ())),
                            preferred_element_type=jnp.float32,
                            precision=_HIGH)
        out_ref[...] = jnp.maximum(y + shift, 0.0)[None]


def _l_call(x2, wlat_t, g2, b2):
    return pl.pallas_call(
        _l_body,
        grid=(2 * B,),
        in_specs=[
            pl.BlockSpec((1, CIN2, M), lambda i: (lax.rem(i, B), 0, 0)),
            pl.BlockSpec((CIN2, COUT), lambda i: (0, 0)),
            pl.BlockSpec((1, COUT), lambda i: (0, 0)),
            pl.BlockSpec((1, COUT), lambda i: (0, 0)),
        ],
        out_specs=pl.BlockSpec(
            (1, M, COUT), lambda i: (jnp.maximum(i - B, 0), 0, 0)),
        out_shape=jax.ShapeDtypeStruct((B, M, COUT), jnp.float32),
        scratch_shapes=[
            pltpu.VMEM((CIN2, CIN2), jnp.float32),
            pltpu.VMEM((CIN2, 1), jnp.float32),
        ],
    )(x2, wlat_t, g2, b2)


def _bf16_round(x):
    u = plsc.bitcast(x, jnp.uint32)
    r = (u + jnp.uint32(0x7FFF) + ((u >> jnp.uint32(16)) & jnp.uint32(1)))
    r = r & jnp.uint32(0xFFFF0000)
    return plsc.bitcast(r, jnp.float32)


def _sc_body(p1x, p1y, p1z, p2x, p2y, p2z, h_hbm, l_hbm, out_hbm,
             cx, cy, cz, cn, fxr, fyr, fzr,
             wb0, wb1, wb2, ib0, ib1, ib2,
             rows0, rows1, rows2, acc, sem0, sem1, sem2):
    nc = 2
    wid = lax.axis_index("s") * nc + lax.axis_index("c")
    b = wid // (NW // B)
    mbase = b * M + (wid % (NW // B)) * MPW
    grow0 = wid * MPW

    pltpu.sync_copy(p1x.at[pl.ds(b * N, N)], cx)
    pltpu.sync_copy(p1y.at[pl.ds(b * N, N)], cy)
    pltpu.sync_copy(p1z.at[pl.ds(b * N, N)], cz)
    pltpu.sync_copy(p2x.at[pl.ds(mbase, MPW)], fxr)
    pltpu.sync_copy(p2y.at[pl.ds(mbase, MPW)], fyr)
    pltpu.sync_copy(p2z.at[pl.ds(mbase, MPW)], fzr)

    def _cn_body(i, _):
        sl = pl.ds(i * 16, 16)
        vx = cx[sl]
        vy = cy[sl]
        vz = cz[sl]
        cn[sl] = vx * vx + vy * vy + vz * vz
        cx[sl] = _bf16_round(vx) * 2.0
        cy[sl] = _bf16_round(vy) * 2.0
        cz[sl] = _bf16_round(vz) * 2.0
        return 0

    lax.fori_loop(0, N // 16, _cn_body, 0)

    big = jnp.full((16,), 3.0e38, jnp.float32)
    zero_i = jnp.zeros((16,), jnp.int32)

    def _group(g, _):
        sl = pl.ds(g * 16, 16)
        fx = fxr[sl]
        fy = fyr[sl]
        fz = fzr[sl]
        fn = fx * fx + fy * fy + fz * fz
        fx = _bf16_round(fx)
        fy = _bf16_round(fy)
        fz = _bf16_round(fz)

        kbig = jnp.full((16,), 0xFFFFFFFF, jnp.uint32)
        hi = jnp.full((16,), 0x80000000, jnp.uint32)
        msk = jnp.full((16,), 0xFFFFFC00, jnp.uint32)

        def _scan(n, carry):
            k1, k2, k3 = carry
            nv = jnp.broadcast_to(n, (16,)).astype(jnp.int32)
            cxv = plsc.load_gather(cx, [nv])
            cyv = plsc.load_gather(cy, [nv])
            czv = plsc.load_gather(cz, [nv])
            cnv = plsc.load_gather(cn, [nv])
            dot2 = fx * cxv + fy * cyv + fz * czv
            d = (fn + cnv) - dot2
            di = plsc.bitcast(d, jnp.int32)
            sgn = plsc.bitcast(di >> 31, jnp.uint32)
            key = plsc.bitcast(di, jnp.uint32) ^ (sgn | hi)
            key = (key & msk) | plsc.bitcast(nv, jnp.uint32)
            a = jnp.minimum(key, k1)
            bq = jnp.maximum(key, k1)
            c = jnp.minimum(bq, k2)
            e = jnp.maximum(bq, k2)
            return a, c, jnp.minimum(e, k3)

        k1, k2, k3 = lax.fori_loop(
            0, N, _scan, (kbig, kbig, kbig), unroll=8)

        base = jnp.broadcast_to(b * N, (16,)).astype(jnp.int32)
        idxm = jnp.full((16,), 0x3FF, jnp.uint32)
        rs = []
        for t, kk in enumerate((k1, k2, k3)):
            ik = plsc.bitcast(kk & idxm, jnp.int32)
            cxv = plsc.load_gather(cx, [ik])
            cyv = plsc.load_gather(cy, [ik])
            czv = plsc.load_gather(cz, [ik])
            cnv = plsc.load_gather(cn, [ik])
            dk = (fn + cnv) - (fx * cxv + fy * cyv + fz * czv)
            rs.append(1.0 / (dk + 1e-8))
            [ib0, ib1, ib2][t][sl] = ik + base
        s = 1.0 / (rs[0] + rs[1] + rs[2])
        wb0[sl] = rs[0] * s
        wb1[sl] = rs[1] * s
        wb2[sl] = rs[2] * s
        return 0

    lax.fori_loop(0, NG, _group, 0)

    rows = [rows0, rows1, rows2]
    sems = [sem0, sem1, sem2]
    ibufs = [ib0, ib1, ib2]
    chunk = 128
    for c in range(MPW // chunk):
        cps = []
        for k in range(3):
            cp = pltpu.make_async_copy(
                h_hbm.at[ibufs[k].at[pl.ds(c * chunk, chunk)]], rows[k],
                sems[k])
            cp.start()
            cps.append(cp)
        pltpu.sync_copy(
            l_hbm.at[pl.ds((grow0 + c * chunk) * COUT, chunk * COUT)], acc)
        for cp in cps:
            cp.wait()

        def _mrow(m, _):
            mv = jnp.broadcast_to(c * chunk + m, (16,)).astype(jnp.int32)
            w0 = plsc.load_gather(wb0, [mv])
            w1 = plsc.load_gather(wb1, [mv])
            w2 = plsc.load_gather(wb2, [mv])
            for q in range(8):
                csl = pl.ds(q * 16, 16)
                g0 = rows0[m, csl]
                g1 = rows1[m, csl]
                g2 = rows2[m, csl]
                sl = pl.ds(m * COUT + q * 16, 16)
                acc[sl] = acc[sl] + w0 * g0 + w1 * g1 + w2 * g2
            return 0

        lax.fori_loop(0, chunk, _mrow, 0)
        pltpu.sync_copy(
            acc, out_hbm.at[pl.ds((grow0 + c * chunk) * COUT, chunk * COUT)])


def _sc_call(p1f, p2f, h_rows, l_flat):
    mesh = plsc.VectorSubcoreMesh(core_axis_name="c", subcore_axis_name="s")
    fn = pl.kernel(
        _sc_body,
        out_type=jax.ShapeDtypeStruct((B * M * COUT,), jnp.float32),
        mesh=mesh,
        compiler_params=pltpu.CompilerParams(
            use_tc_tiling_on_sc=False, needs_layout_passes=False),
        scratch_types=[
            pltpu.VMEM((N,), jnp.float32),
            pltpu.VMEM((N,), jnp.float32),
            pltpu.VMEM((N,), jnp.float32),
            pltpu.VMEM((N,), jnp.float32),
            pltpu.VMEM((MPW,), jnp.float32),
            pltpu.VMEM((MPW,), jnp.float32),
            pltpu.VMEM((MPW,), jnp.float32),
            pltpu.VMEM((MPW,), jnp.float32),
            pltpu.VMEM((MPW,), jnp.float32),
            pltpu.VMEM((MPW,), jnp.float32),
            pltpu.VMEM((MPW,), jnp.int32),
            pltpu.VMEM((MPW,), jnp.int32),
            pltpu.VMEM((MPW,), jnp.int32),
            pltpu.VMEM((128, COUT), jnp.float32),
            pltpu.VMEM((128, COUT), jnp.float32),
            pltpu.VMEM((128, COUT), jnp.float32),
            pltpu.VMEM((128 * COUT,), jnp.float32),
            pltpu.SemaphoreType.DMA,
            pltpu.SemaphoreType.DMA,
            pltpu.SemaphoreType.DMA,
        ],
    )
    return fn(p1f[0], p1f[1], p1f[2], p2f[0], p2f[1], p2f[2], h_rows, l_flat)


@jax.jit
def kernel(x1, p1, x2, p2, W_up, gamma1, beta1, W_lat, gamma2, beta2):
    p1f = jnp.transpose(p1, (2, 0, 1)).reshape(3, B * N)
    p2f = jnp.transpose(p2, (2, 0, 1)).reshape(3, B * M)
    h = _h_call(x1, W_up.T, gamma1.reshape(1, COUT), beta1.reshape(1, COUT))
    l = _l_call(x2, W_lat.T, gamma2.reshape(1, COUT), beta2.reshape(1, COUT))
    out_flat = _sc_call(p1f, p2f, h.reshape(B * N, COUT),
                        l.reshape(B * M * COUT))
    out = jnp.transpose(out_flat.reshape(B, M, COUT), (0, 2, 1))
    return (out, p2)


# speedup vs baseline: 1.0838x; 1.0119x over previous
import functools

import jax
import jax.numpy as jnp
from jax import lax
from jax.experimental import pallas as pl
from jax.experimental.pallas import tpu as pltpu
from jax.experimental.pallas import tpu_sc as plsc

B, N, M, CIN, COUT = 4, 1024, 4096, 256, 128
CIN2 = COUT
NW = 32
MPW = B * M // NW
NG = MPW // 16
BN_EPS = 1e-5
_HIGH = jax.lax.Precision.HIGHEST


def _h_body(x1_ref, wt_ref, g_ref, b_ref, out_ref):
    wt = wt_ref[...]
    ys = [
        lax.dot_general(x1_ref[b], wt, (((0,), (0,)), ((), ())),
                        preferred_element_type=jnp.float32, precision=_HIGH)
        for b in range(B)
    ]
    tot = ys[0] + ys[1] + ys[2] + ys[3]
    mean = jnp.sum(tot, axis=0, keepdims=True) * (1.0 / (B * N))
    sq = sum(jnp.sum(y * y, axis=0, keepdims=True) for y in ys)
    var = sq * (1.0 / (B * N)) - mean * mean
    scale = g_ref[...] / jnp.sqrt(var + BN_EPS)
    shift = b_ref[...] - mean * scale
    for b in range(B):
        out_ref[b] = jnp.maximum(ys[b] * scale + shift, 0.0)


def _h_call(x1, wup_t, g1, b1):
    return pl.pallas_call(
        _h_body,
        out_shape=jax.ShapeDtypeStruct((B, N, COUT), jnp.float32),
    )(x1, wup_t, g1, b1)


def _l_body(x2_ref, wt_ref, g_ref, b_ref, out_ref, cov_ref, mu_ref):
    i = pl.program_id(0)

    @pl.when(i == 0)
    def _init():
        cov_ref[...] = jnp.zeros((CIN2, CIN2), jnp.float32)
        mu_ref[...] = jnp.zeros((CIN2, 1), jnp.float32)

    xb = x2_ref[0]

    @pl.when(i < B)
    def _accum():
        cov_ref[...] += lax.dot_general(
            xb, xb, (((1,), (1,)), ((), ())),
            preferred_element_type=jnp.float32, precision=_HIGH)
        mu_ref[...] += jnp.sum(xb, axis=1, keepdims=True)
        out_ref[...] = jnp.zeros_like(out_ref)

    @pl.when(i >= B)
    def _emit():
        wt = wt_ref[...]
        inv_l = 1.0 / (B * M)
        mean = lax.dot_general(mu_ref[...], wt, (((0,), (0,)), ((), ())),
                               preferred_element_type=jnp.float32,
                               precision=_HIGH) * inv_l
        t = lax.dot_general(cov_ref[...], wt, (((1,), (0,)), ((), ())),
                            preferred_element_type=jnp.float32,
                            precision=_HIGH)
        e2 = jnp.sum(t * wt, axis=0, keepdims=True) * inv_l
        var = e2 - mean * mean
        scale = g_ref[...] / jnp.sqrt(var + BN_EPS)
        shift = b_ref[...] - mean * scale
        wf = wt * scale
        y = lax.dot_general(xb, wf, (((0,), (0,)), ((), ())),
                            preferred_element_type=jnp.float32,
                            precision=_HIGH)
        out_ref[...] = jnp.maximum(y + shift, 0.0)[None]


def _l_call(x2, wlat_t, g2, b2):
    return pl.pallas_call(
        _l_body,
        grid=(2 * B,),
        in_specs=[
            pl.BlockSpec((1, CIN2, M), lambda i: (lax.rem(i, B), 0, 0)),
            pl.BlockSpec((CIN2, COUT), lambda i: (0, 0)),
            pl.BlockSpec((1, COUT), lambda i: (0, 0)),
            pl.BlockSpec((1, COUT), lambda i: (0, 0)),
        ],
        out_specs=pl.BlockSpec(
            (1, M, COUT), lambda i: (jnp.maximum(i - B, 0), 0, 0)),
        out_shape=jax.ShapeDtypeStruct((B, M, COUT), jnp.float32),
        scratch_shapes=[
            pltpu.VMEM((CIN2, CIN2), jnp.float32),
            pltpu.VMEM((CIN2, 1), jnp.float32),
        ],
    )(x2, wlat_t, g2, b2)


def _bf16_round(x):
    u = plsc.bitcast(x, jnp.uint32)
    r = (u + jnp.uint32(0x7FFF) + ((u >> jnp.uint32(16)) & jnp.uint32(1)))
    r = r & jnp.uint32(0xFFFF0000)
    return plsc.bitcast(r, jnp.float32)


def _sc_body(p1x, p1y, p1z, p2x, p2y, p2z, h_hbm, l_hbm, out_hbm,
             cx, cy, cz, cn, fxr, fyr, fzr,
             wb0, wb1, wb2, ib0, ib1, ib2,
             rows0, rows1, rows2, acc, sem0, sem1, sem2):
    nc = 2
    wid = lax.axis_index("s") * nc + lax.axis_index("c")
    b = wid // (NW // B)
    mbase = b * M + (wid % (NW // B)) * MPW
    grow0 = wid * MPW

    pltpu.sync_copy(p1x.at[pl.ds(b * N, N)], cx)
    pltpu.sync_copy(p1y.at[pl.ds(b * N, N)], cy)
    pltpu.sync_copy(p1z.at[pl.ds(b * N, N)], cz)
    pltpu.sync_copy(p2x.at[pl.ds(mbase, MPW)], fxr)
    pltpu.sync_copy(p2y.at[pl.ds(mbase, MPW)], fyr)
    pltpu.sync_copy(p2z.at[pl.ds(mbase, MPW)], fzr)

    def _cn_body(i, _):
        sl = pl.ds(i * 16, 16)
        vx = cx[sl]
        vy = cy[sl]
        vz = cz[sl]
        cn[sl] = vx * vx + vy * vy + vz * vz
        cx[sl] = _bf16_round(vx) * 2.0
        cy[sl] = _bf16_round(vy) * 2.0
        cz[sl] = _bf16_round(vz) * 2.0
        return 0

    lax.fori_loop(0, N // 16, _cn_body, 0)

    big = jnp.full((16,), 3.0e38, jnp.float32)
    zero_i = jnp.zeros((16,), jnp.int32)

    kbig = jnp.full((16,), 0xFFFFFFFF, jnp.uint32)
    hi = jnp.full((16,), 0x80000000, jnp.uint32)
    msk = jnp.full((16,), 0xFFFFFC00, jnp.uint32)
    idxm = jnp.full((16,), 0x3FF, jnp.uint32)

    def _group(gp, _):
        sls = [pl.ds(gp * 32, 16), pl.ds(gp * 32 + 16, 16)]
        fxs, fys, fzs, fns = [], [], [], []
        for sl in sls:
            fx = fxr[sl]
            fy = fyr[sl]
            fz = fzr[sl]
            fns.append(fx * fx + fy * fy + fz * fz)
            fxs.append(_bf16_round(fx))
            fys.append(_bf16_round(fy))
            fzs.append(_bf16_round(fz))

        def _scan(n, carry):
            nv = jnp.broadcast_to(n, (16,)).astype(jnp.int32)
            nvu = plsc.bitcast(nv, jnp.uint32)
            cxv = plsc.load_gather(cx, [nv])
            cyv = plsc.load_gather(cy, [nv])
            czv = plsc.load_gather(cz, [nv])
            cnv = plsc.load_gather(cn, [nv])
            out = []
            for t in range(2):
                k1, k2, k3 = carry[3 * t:3 * t + 3]
                dot2 = fxs[t] * cxv + fys[t] * cyv + fzs[t] * czv
                d = (fns[t] + cnv) - dot2
                di = plsc.bitcast(d, jnp.int32)
                sgn = plsc.bitcast(di >> 31, jnp.uint32)
                key = plsc.bitcast(di, jnp.uint32) ^ (sgn | hi)
                key = (key & msk) | nvu
                a = jnp.minimum(key, k1)
                bq = jnp.maximum(key, k1)
                c = jnp.minimum(bq, k2)
                e = jnp.maximum(bq, k2)
                out += [a, c, jnp.minimum(e, k3)]
            return tuple(out)

        res = lax.fori_loop(0, N, _scan, (kbig,) * 6, unroll=4)

        base = jnp.broadcast_to(b * N, (16,)).astype(jnp.int32)
        for t, sl in enumerate(sls):
            rs = []
            for j in range(3):
                kk = res[3 * t + j]
                ik = plsc.bitcast(kk & idxm, jnp.int32)
                cxv = plsc.load_gather(cx, [ik])
                cyv = plsc.load_gather(cy, [ik])
                czv = plsc.load_gather(cz, [ik])
                cnv = plsc.load_gather(cn, [ik])
                dk = (fns[t] + cnv) - (fxs[t] * cxv + fys[t] * cyv
                                       + fzs[t] * czv)
                rs.append(1.0 / (dk + 1e-8))
                [ib0, ib1, ib2][j][sl] = ik + base
            s = 1.0 / (rs[0] + rs[1] + rs[2])
            wb0[sl] = rs[0] * s
            wb1[sl] = rs[1] * s
            wb2[sl] = rs[2] * s
        return 0

    lax.fori_loop(0, NG // 2, _group, 0)

    rows = [rows0, rows1, rows2]
    sems = [sem0, sem1, sem2]
    ibufs = [ib0, ib1, ib2]
    chunk = 128
    for c in range(MPW // chunk):
        cps = []
        for k in range(3):
            cp = pltpu.make_async_copy(
                h_hbm.at[ibufs[k].at[pl.ds(c * chunk, chunk)]], rows[k],
                sems[k])
            cp.start()
            cps.append(cp)
        pltpu.sync_copy(
            l_hbm.at[pl.ds((grow0 + c * chunk) * COUT, chunk * COUT)], acc)
        for cp in cps:
            cp.wait()

        def _mrow(m, _):
            mv = jnp.broadcast_to(c * chunk + m, (16,)).astype(jnp.int32)
            w0 = plsc.load_gather(wb0, [mv])
            w1 = plsc.load_gather(wb1, [mv])
            w2 = plsc.load_gather(wb2, [mv])
            for q in range(8):
                csl = pl.ds(q * 16, 16)
                g0 = rows0[m, csl]
                g1 = rows1[m, csl]
                g2 = rows2[m, csl]
                sl = pl.ds(m * COUT + q * 16, 16)
                acc[sl] = acc[sl] + w0 * g0 + w1 * g1 + w2 * g2
            return 0

        lax.fori_loop(0, chunk, _mrow, 0)
        pltpu.sync_copy(
            acc, out_hbm.at[pl.ds((grow0 + c * chunk) * COUT, chunk * COUT)])


def _sc_call(p1f, p2f, h_rows, l_flat):
    mesh = plsc.VectorSubcoreMesh(core_axis_name="c", subcore_axis_name="s")
    fn = pl.kernel(
        _sc_body,
        out_type=jax.ShapeDtypeStruct((B * M * COUT,), jnp.float32),
        mesh=mesh,
        compiler_params=pltpu.CompilerParams(
            use_tc_tiling_on_sc=False, needs_layout_passes=False),
        scratch_types=[
            pltpu.VMEM((N,), jnp.float32),
            pltpu.VMEM((N,), jnp.float32),
            pltpu.VMEM((N,), jnp.float32),
            pltpu.VMEM((N,), jnp.float32),
            pltpu.VMEM((MPW,), jnp.float32),
            pltpu.VMEM((MPW,), jnp.float32),
            pltpu.VMEM((MPW,), jnp.float32),
            pltpu.VMEM((MPW,), jnp.float32),
            pltpu.VMEM((MPW,), jnp.float32),
            pltpu.VMEM((MPW,), jnp.float32),
            pltpu.VMEM((MPW,), jnp.int32),
            pltpu.VMEM((MPW,), jnp.int32),
            pltpu.VMEM((MPW,), jnp.int32),
            pltpu.VMEM((128, COUT), jnp.float32),
            pltpu.VMEM((128, COUT), jnp.float32),
            pltpu.VMEM((128, COUT), jnp.float32),
            pltpu.VMEM((128 * COUT,), jnp.float32),
            pltpu.SemaphoreType.DMA,
            pltpu.SemaphoreType.DMA,
            pltpu.SemaphoreType.DMA,
        ],
    )
    return fn(p1f[0], p1f[1], p1f[2], p2f[0], p2f[1], p2f[2], h_rows, l_flat)


@jax.jit
def kernel(x1, p1, x2, p2, W_up, gamma1, beta1, W_lat, gamma2, beta2):
    p1f = jnp.transpose(p1, (2, 0, 1)).reshape(3, B * N)
    p2f = jnp.transpose(p2, (2, 0, 1)).reshape(3, B * M)
    h = _h_call(x1, W_up.T, gamma1.reshape(1, COUT), beta1.reshape(1, COUT))
    l = _l_call(x2, W_lat.T, gamma2.reshape(1, COUT), beta2.reshape(1, COUT))
    out_flat = _sc_call(p1f, p2f, h.reshape(B * N, COUT),
                        l.reshape(B * M * COUT))
    out = jnp.transpose(out_flat.reshape(B, M, COUT), (0, 2, 1))
    return (out, p2)


# speedup vs baseline: 1.0934x; 1.0089x over previous
import functools

import jax
import jax.numpy as jnp
from jax import lax
from jax.experimental import pallas as pl
from jax.experimental.pallas import tpu as pltpu
from jax.experimental.pallas import tpu_sc as plsc

B, N, M, CIN, COUT = 4, 1024, 4096, 256, 128
CIN2 = COUT
NW = 32
MPW = B * M // NW
NG = MPW // 16
BN_EPS = 1e-5
_HIGH = jax.lax.Precision.HIGHEST


def _h_body(x1_ref, wt_ref, g_ref, b_ref, out_ref):
    wt = wt_ref[...]
    ys = [
        lax.dot_general(x1_ref[b], wt, (((0,), (0,)), ((), ())),
                        preferred_element_type=jnp.float32, precision=_HIGH)
        for b in range(B)
    ]
    tot = ys[0] + ys[1] + ys[2] + ys[3]
    mean = jnp.sum(tot, axis=0, keepdims=True) * (1.0 / (B * N))
    sq = sum(jnp.sum(y * y, axis=0, keepdims=True) for y in ys)
    var = sq * (1.0 / (B * N)) - mean * mean
    scale = g_ref[...] / jnp.sqrt(var + BN_EPS)
    shift = b_ref[...] - mean * scale
    for b in range(B):
        out_ref[b] = jnp.maximum(ys[b] * scale + shift, 0.0)


def _h_call(x1, wup_t, g1, b1):
    return pl.pallas_call(
        _h_body,
        out_shape=jax.ShapeDtypeStruct((B, N, COUT), jnp.float32),
    )(x1, wup_t, g1, b1)


def _l_body(x2_ref, wt_ref, g_ref, b_ref, out_ref, cov_ref, mu_ref):
    i = pl.program_id(0)

    @pl.when(i == 0)
    def _init():
        cov_ref[...] = jnp.zeros((CIN2, CIN2), jnp.float32)
        mu_ref[...] = jnp.zeros((CIN2, 1), jnp.float32)

    xb = x2_ref[0]

    @pl.when(i < B)
    def _accum():
        cov_ref[...] += lax.dot_general(
            xb, xb, (((1,), (1,)), ((), ())),
            preferred_element_type=jnp.float32, precision=_HIGH)
        mu_ref[...] += jnp.sum(xb, axis=1, keepdims=True)
        out_ref[...] = jnp.zeros_like(out_ref)

    @pl.when(i >= B)
    def _emit():
        wt = wt_ref[...]
        inv_l = 1.0 / (B * M)
        mean = lax.dot_general(mu_ref[...], wt, (((0,), (0,)), ((), ())),
                               preferred_element_type=jnp.float32,
                               precision=_HIGH) * inv_l
        t = lax.dot_general(cov_ref[...], wt, (((1,), (0,)), ((), ())),
                            preferred_element_type=jnp.float32,
                            precision=_HIGH)
        e2 = jnp.sum(t * wt, axis=0, keepdims=True) * inv_l
        var = e2 - mean * mean
        scale = g_ref[...] / jnp.sqrt(var + BN_EPS)
        shift = b_ref[...] - mean * scale
        wf = wt * scale
        y = lax.dot_general(xb, wf, (((0,), (0,)), ((), ())),
                            preferred_element_type=jnp.float32,
                            precision=_HIGH)
        out_ref[...] = jnp.maximum(y + shift, 0.0)[None]


def _l_call(x2, wlat_t, g2, b2):
    return pl.pallas_call(
        _l_body,
        grid=(2 * B,),
        in_specs=[
            pl.BlockSpec((1, CIN2, M), lambda i: (lax.rem(i, B), 0, 0)),
            pl.BlockSpec((CIN2, COUT), lambda i: (0, 0)),
            pl.BlockSpec((1, COUT), lambda i: (0, 0)),
            pl.BlockSpec((1, COUT), lambda i: (0, 0)),
        ],
        out_specs=pl.BlockSpec(
            (1, M, COUT), lambda i: (jnp.maximum(i - B, 0), 0, 0)),
        out_shape=jax.ShapeDtypeStruct((B, M, COUT), jnp.float32),
        scratch_shapes=[
            pltpu.VMEM((CIN2, CIN2), jnp.float32),
            pltpu.VMEM((CIN2, 1), jnp.float32),
        ],
    )(x2, wlat_t, g2, b2)


def _bf16_round(x):
    u = plsc.bitcast(x, jnp.uint32)
    r = (u + jnp.uint32(0x7FFF) + ((u >> jnp.uint32(16)) & jnp.uint32(1)))
    r = r & jnp.uint32(0xFFFF0000)
    return plsc.bitcast(r, jnp.float32)


def _sc_body(p1x, p1y, p1z, p2x, p2y, p2z, h_hbm, l_hbm, out_hbm,
             cx, cy, cz, cn, fxr, fyr, fzr,
             wb0, wb1, wb2, ib0, ib1, ib2,
             rows0, rows1, rows2, acc, sem0, sem1, sem2):
    nc = 2
    wid = lax.axis_index("s") * nc + lax.axis_index("c")
    b = wid // (NW // B)
    mbase = b * M + (wid % (NW // B)) * MPW
    grow0 = wid * MPW

    pltpu.sync_copy(p1x.at[pl.ds(b * N, N)], cx)
    pltpu.sync_copy(p1y.at[pl.ds(b * N, N)], cy)
    pltpu.sync_copy(p1z.at[pl.ds(b * N, N)], cz)
    pltpu.sync_copy(p2x.at[pl.ds(mbase, MPW)], fxr)
    pltpu.sync_copy(p2y.at[pl.ds(mbase, MPW)], fyr)
    pltpu.sync_copy(p2z.at[pl.ds(mbase, MPW)], fzr)

    def _cn_body(i, _):
        sl = pl.ds(i * 16, 16)
        vx = cx[sl]
        vy = cy[sl]
        vz = cz[sl]
        cn[sl] = vx * vx + vy * vy + vz * vz
        cx[sl] = _bf16_round(vx) * 2.0
        cy[sl] = _bf16_round(vy) * 2.0
        cz[sl] = _bf16_round(vz) * 2.0
        return 0

    lax.fori_loop(0, N // 16, _cn_body, 0)

    big = jnp.full((16,), 3.0e38, jnp.float32)
    zero_i = jnp.zeros((16,), jnp.int32)

    kbig = jnp.full((16,), 0xFFFFFFFF, jnp.uint32)
    hi = jnp.full((16,), 0x80000000, jnp.uint32)
    msk = jnp.full((16,), 0xFFFFFC00, jnp.uint32)
    idxm = jnp.full((16,), 0x3FF, jnp.uint32)

    def _group(gp, _):
        sls = [pl.ds(gp * 64 + 16 * t, 16) for t in range(4)]
        fxs, fys, fzs, fns = [], [], [], []
        for sl in sls:
            fx = fxr[sl]
            fy = fyr[sl]
            fz = fzr[sl]
            fns.append(fx * fx + fy * fy + fz * fz)
            fxs.append(_bf16_round(fx))
            fys.append(_bf16_round(fy))
            fzs.append(_bf16_round(fz))

        def _scan(n, carry):
            nv = jnp.broadcast_to(n, (16,)).astype(jnp.int32)
            nvu = plsc.bitcast(nv, jnp.uint32)
            cxv = plsc.load_gather(cx, [nv])
            cyv = plsc.load_gather(cy, [nv])
            czv = plsc.load_gather(cz, [nv])
            cnv = plsc.load_gather(cn, [nv])
            out = []
            for t in range(4):
                k1, k2, k3 = carry[3 * t:3 * t + 3]
                dot2 = fxs[t] * cxv + fys[t] * cyv + fzs[t] * czv
                d = (fns[t] + cnv) - dot2
                di = plsc.bitcast(d, jnp.int32)
                sgn = plsc.bitcast(di >> 31, jnp.uint32)
                key = plsc.bitcast(di, jnp.uint32) ^ (sgn | hi)
                key = (key & msk) | nvu
                a = jnp.minimum(key, k1)
                bq = jnp.maximum(key, k1)
                c = jnp.minimum(bq, k2)
                e = jnp.maximum(bq, k2)
                out += [a, c, jnp.minimum(e, k3)]
            return tuple(out)

        res = lax.fori_loop(0, N, _scan, (kbig,) * 12, unroll=2)

        base = jnp.broadcast_to(b * N, (16,)).astype(jnp.int32)
        for t, sl in enumerate(sls):
            rs = []
            for j in range(3):
                kk = res[3 * t + j]
                ik = plsc.bitcast(kk & idxm, jnp.int32)
                cxv = plsc.load_gather(cx, [ik])
                cyv = plsc.load_gather(cy, [ik])
                czv = plsc.load_gather(cz, [ik])
                cnv = plsc.load_gather(cn, [ik])
                dk = (fns[t] + cnv) - (fxs[t] * cxv + fys[t] * cyv
                                       + fzs[t] * czv)
                rs.append(1.0 / (dk + 1e-8))
                [ib0, ib1, ib2][j][sl] = ik + base
            s = 1.0 / (rs[0] + rs[1] + rs[2])
            wb0[sl] = rs[0] * s
            wb1[sl] = rs[1] * s
            wb2[sl] = rs[2] * s
        return 0

    lax.fori_loop(0, NG // 4, _group, 0)

    rows = [rows0, rows1, rows2]
    sems = [sem0, sem1, sem2]
    ibufs = [ib0, ib1, ib2]
    chunk = 128
    for c in range(MPW // chunk):
        cps = []
        for k in range(3):
            cp = pltpu.make_async_copy(
                h_hbm.at[ibufs[k].at[pl.ds(c * chunk, chunk)]], rows[k],
                sems[k])
            cp.start()
            cps.append(cp)
        pltpu.sync_copy(
            l_hbm.at[pl.ds((grow0 + c * chunk) * COUT, chunk * COUT)], acc)
        for cp in cps:
            cp.wait()

        def _mrow(m, _):
            mv = jnp.broadcast_to(c * chunk + m, (16,)).astype(jnp.int32)
            w0 = plsc.load_gather(wb0, [mv])
            w1 = plsc.load_gather(wb1, [mv])
            w2 = plsc.load_gather(wb2, [mv])
            for q in range(8):
                csl = pl.ds(q * 16, 16)
                g0 = rows0[m, csl]
                g1 = rows1[m, csl]
                g2 = rows2[m, csl]
                sl = pl.ds(m * COUT + q * 16, 16)
                acc[sl] = acc[sl] + w0 * g0 + w1 * g1 + w2 * g2
            return 0

        lax.fori_loop(0, chunk, _mrow, 0)
        pltpu.sync_copy(
            acc, out_hbm.at[pl.ds((grow0 + c * chunk) * COUT, chunk * COUT)])


def _sc_call(p1f, p2f, h_rows, l_flat):
    mesh = plsc.VectorSubcoreMesh(core_axis_name="c", subcore_axis_name="s")
    fn = pl.kernel(
        _sc_body,
        out_type=jax.ShapeDtypeStruct((B * M * COUT,), jnp.float32),
        mesh=mesh,
        compiler_params=pltpu.CompilerParams(
            use_tc_tiling_on_sc=False, needs_layout_passes=False),
        scratch_types=[
            pltpu.VMEM((N,), jnp.float32),
            pltpu.VMEM((N,), jnp.float32),
            pltpu.VMEM((N,), jnp.float32),
            pltpu.VMEM((N,), jnp.float32),
            pltpu.VMEM((MPW,), jnp.float32),
            pltpu.VMEM((MPW,), jnp.float32),
            pltpu.VMEM((MPW,), jnp.float32),
            pltpu.VMEM((MPW,), jnp.float32),
            pltpu.VMEM((MPW,), jnp.float32),
            pltpu.VMEM((MPW,), jnp.float32),
            pltpu.VMEM((MPW,), jnp.int32),
            pltpu.VMEM((MPW,), jnp.int32),
            pltpu.VMEM((MPW,), jnp.int32),
            pltpu.VMEM((128, COUT), jnp.float32),
            pltpu.VMEM((128, COUT), jnp.float32),
            pltpu.VMEM((128, COUT), jnp.float32),
            pltpu.VMEM((128 * COUT,), jnp.float32),
            pltpu.SemaphoreType.DMA,
            pltpu.SemaphoreType.DMA,
            pltpu.SemaphoreType.DMA,
        ],
    )
    return fn(p1f[0], p1f[1], p1f[2], p2f[0], p2f[1], p2f[2], h_rows, l_flat)


@jax.jit
def kernel(x1, p1, x2, p2, W_up, gamma1, beta1, W_lat, gamma2, beta2):
    p1f = jnp.transpose(p1, (2, 0, 1)).reshape(3, B * N)
    p2f = jnp.transpose(p2, (2, 0, 1)).reshape(3, B * M)
    h = _h_call(x1, W_up.T, gamma1.reshape(1, COUT), beta1.reshape(1, COUT))
    l = _l_call(x2, W_lat.T, gamma2.reshape(1, COUT), beta2.reshape(1, COUT))
    out_flat = _sc_call(p1f, p2f, h.reshape(B * N, COUT),
                        l.reshape(B * M * COUT))
    out = jnp.transpose(out_flat.reshape(B, M, COUT), (0, 2, 1))
    return (out, p2)


# speedup vs baseline: 1.0941x; 1.0006x over previous
import functools

import jax
import jax.numpy as jnp
from jax import lax
from jax.experimental import pallas as pl
from jax.experimental.pallas import tpu as pltpu
from jax.experimental.pallas import tpu_sc as plsc

B, N, M, CIN, COUT = 4, 1024, 4096, 256, 128
CIN2 = COUT
NW = 32
MPW = B * M // NW
NG = MPW // 16
BN_EPS = 1e-5
_HIGH = jax.lax.Precision.HIGHEST


def _h_body(x1_ref, wt_ref, g_ref, b_ref, out_ref):
    wt = wt_ref[...]
    ys = [
        lax.dot_general(x1_ref[b], wt, (((0,), (0,)), ((), ())),
                        preferred_element_type=jnp.float32, precision=_HIGH)
        for b in range(B)
    ]
    tot = ys[0] + ys[1] + ys[2] + ys[3]
    mean = jnp.sum(tot, axis=0, keepdims=True) * (1.0 / (B * N))
    sq = sum(jnp.sum(y * y, axis=0, keepdims=True) for y in ys)
    var = sq * (1.0 / (B * N)) - mean * mean
    scale = g_ref[...] / jnp.sqrt(var + BN_EPS)
    shift = b_ref[...] - mean * scale
    for b in range(B):
        out_ref[b] = jnp.maximum(ys[b] * scale + shift, 0.0)


def _h_call(x1, wup_t, g1, b1):
    return pl.pallas_call(
        _h_body,
        out_shape=jax.ShapeDtypeStruct((B, N, COUT), jnp.float32),
    )(x1, wup_t, g1, b1)


def _l_body(x2_ref, wt_ref, g_ref, b_ref, out_ref, cov_ref, mu_ref):
    i = pl.program_id(0)

    @pl.when(i == 0)
    def _init():
        cov_ref[...] = jnp.zeros((CIN2, CIN2), jnp.float32)
        mu_ref[...] = jnp.zeros((CIN2, 1), jnp.float32)

    xb = x2_ref[0]

    @pl.when(i < B)
    def _accum():
        cov_ref[...] += lax.dot_general(
            xb, xb, (((1,), (1,)), ((), ())),
            preferred_element_type=jnp.float32, precision=_HIGH)
        mu_ref[...] += jnp.sum(xb, axis=1, keepdims=True)
        out_ref[...] = jnp.zeros_like(out_ref)

    @pl.when(i >= B)
    def _emit():
        wt = wt_ref[...]
        inv_l = 1.0 / (B * M)
        mean = lax.dot_general(mu_ref[...], wt, (((0,), (0,)), ((), ())),
                               preferred_element_type=jnp.float32,
                               precision=_HIGH) * inv_l
        t = lax.dot_general(cov_ref[...], wt, (((1,), (0,)), ((), ())),
                            preferred_element_type=jnp.float32,
                            precision=_HIGH)
        e2 = jnp.sum(t * wt, axis=0, keepdims=True) * inv_l
        var = e2 - mean * mean
        scale = g_ref[...] / jnp.sqrt(var + BN_EPS)
        shift = b_ref[...] - mean * scale
        wf = wt * scale
        y = lax.dot_general(xb, wf, (((0,), (0,)), ((), ())),
                            preferred_element_type=jnp.float32,
                            precision=_HIGH)
        out_ref[...] = jnp.maximum(y + shift, 0.0)[None]


def _l_call(x2, wlat_t, g2, b2):
    return pl.pallas_call(
        _l_body,
        grid=(2 * B,),
        in_specs=[
            pl.BlockSpec((1, CIN2, M), lambda i: (lax.rem(i, B), 0, 0)),
            pl.BlockSpec((CIN2, COUT), lambda i: (0, 0)),
            pl.BlockSpec((1, COUT), lambda i: (0, 0)),
            pl.BlockSpec((1, COUT), lambda i: (0, 0)),
        ],
        out_specs=pl.BlockSpec(
            (1, M, COUT), lambda i: (jnp.maximum(i - B, 0), 0, 0)),
        out_shape=jax.ShapeDtypeStruct((B, M, COUT), jnp.float32),
        scratch_shapes=[
            pltpu.VMEM((CIN2, CIN2), jnp.float32),
            pltpu.VMEM((CIN2, 1), jnp.float32),
        ],
    )(x2, wlat_t, g2, b2)


def _bf16_round(x):
    u = plsc.bitcast(x, jnp.uint32)
    r = (u + jnp.uint32(0x7FFF) + ((u >> jnp.uint32(16)) & jnp.uint32(1)))
    r = r & jnp.uint32(0xFFFF0000)
    return plsc.bitcast(r, jnp.float32)


def _sc_body(p1x, p1y, p1z, p2x, p2y, p2z, h_hbm, l_hbm, out_hbm,
             cx, cy, cz, cn, fxr, fyr, fzr,
             wb0, wb1, wb2, ib0, ib1, ib2,
             rows0, rows1, rows2, acc, sem0, sem1, sem2):
    nc = 2
    wid = lax.axis_index("s") * nc + lax.axis_index("c")
    b = wid // (NW // B)
    mbase = b * M + (wid % (NW // B)) * MPW
    grow0 = wid * MPW

    pltpu.sync_copy(p1x.at[pl.ds(b * N, N)], cx)
    pltpu.sync_copy(p1y.at[pl.ds(b * N, N)], cy)
    pltpu.sync_copy(p1z.at[pl.ds(b * N, N)], cz)
    pltpu.sync_copy(p2x.at[pl.ds(mbase, MPW)], fxr)
    pltpu.sync_copy(p2y.at[pl.ds(mbase, MPW)], fyr)
    pltpu.sync_copy(p2z.at[pl.ds(mbase, MPW)], fzr)

    def _cn_body(i, _):
        sl = pl.ds(i * 16, 16)
        vx = cx[sl]
        vy = cy[sl]
        vz = cz[sl]
        cn[sl] = vx * vx + vy * vy + vz * vz
        cx[sl] = _bf16_round(vx) * 2.0
        cy[sl] = _bf16_round(vy) * 2.0
        cz[sl] = _bf16_round(vz) * 2.0
        return 0

    lax.fori_loop(0, N // 16, _cn_body, 0)

    big = jnp.full((16,), 3.0e38, jnp.float32)
    zero_i = jnp.zeros((16,), jnp.int32)

    kbig = jnp.full((16,), 0xFFFFFFFF, jnp.uint32)
    hi = jnp.full((16,), 0x80000000, jnp.uint32)
    msk = jnp.full((16,), 0xFFFFFC00, jnp.uint32)
    idxm = jnp.full((16,), 0x3FF, jnp.uint32)

    def _group(gp, _):
        sls = [pl.ds(gp * 64 + 16 * t, 16) for t in range(4)]
        fxs, fys, fzs, fns = [], [], [], []
        for sl in sls:
            fx = fxr[sl]
            fy = fyr[sl]
            fz = fzr[sl]
            fns.append(fx * fx + fy * fy + fz * fz)
            fxs.append(_bf16_round(fx))
            fys.append(_bf16_round(fy))
            fzs.append(_bf16_round(fz))

        def _scan(n, carry):
            nv = jnp.broadcast_to(n, (16,)).astype(jnp.int32)
            nvu = plsc.bitcast(nv, jnp.uint32)
            cxv = plsc.load_gather(cx, [nv])
            cyv = plsc.load_gather(cy, [nv])
            czv = plsc.load_gather(cz, [nv])
            cnv = plsc.load_gather(cn, [nv])
            out = []
            for t in range(4):
                k1, k2, k3 = carry[3 * t:3 * t + 3]
                dot2 = fxs[t] * cxv + fys[t] * cyv + fzs[t] * czv
                d = (fns[t] + cnv) - dot2
                di = plsc.bitcast(d, jnp.int32)
                sgn = plsc.bitcast(di >> 31, jnp.uint32)
                key = plsc.bitcast(di, jnp.uint32) ^ (sgn | hi)
                key = (key & msk) | nvu
                a = jnp.minimum(key, k1)
                bq = jnp.maximum(key, k1)
                c = jnp.minimum(bq, k2)
                e = jnp.maximum(bq, k2)
                out += [a, c, jnp.minimum(e, k3)]
            return tuple(out)

        res = lax.fori_loop(0, N, _scan, (kbig,) * 12, unroll=2)

        base = jnp.broadcast_to(b * N, (16,)).astype(jnp.int32)
        for t, sl in enumerate(sls):
            rs = []
            for j in range(3):
                kk = res[3 * t + j]
                ik = plsc.bitcast(kk & idxm, jnp.int32)
                cxv = plsc.load_gather(cx, [ik])
                cyv = plsc.load_gather(cy, [ik])
                czv = plsc.load_gather(cz, [ik])
                cnv = plsc.load_gather(cn, [ik])
                dk = (fns[t] + cnv) - (fxs[t] * cxv + fys[t] * cyv
                                       + fzs[t] * czv)
                rs.append(1.0 / (dk + 1e-8))
                [ib0, ib1, ib2][j][sl] = ik + base
            s = 1.0 / (rs[0] + rs[1] + rs[2])
            wb0[sl] = rs[0] * s
            wb1[sl] = rs[1] * s
            wb2[sl] = rs[2] * s
        return 0

    lax.fori_loop(0, NG // 4, _group, 0)

    rows = [rows0, rows1, rows2]
    sems = [sem0, sem1, sem2]
    ibufs = [ib0, ib1, ib2]
    chunk = 128
    for c in range(MPW // chunk):
        cps = []
        for k in range(3):
            cp = pltpu.make_async_copy(
                h_hbm.at[ibufs[k].at[pl.ds(c * chunk, chunk)]], rows[k],
                sems[k])
            cp.start()
            cps.append(cp)
        pltpu.sync_copy(
            l_hbm.at[pl.ds((grow0 + c * chunk) * COUT, chunk * COUT)], acc)
        for cp in cps:
            cp.wait()

        def _mrow(m, _):
            mv = jnp.broadcast_to(c * chunk + m, (16,)).astype(jnp.int32)
            w0 = plsc.load_gather(wb0, [mv])
            w1 = plsc.load_gather(wb1, [mv])
            w2 = plsc.load_gather(wb2, [mv])
            for q in range(8):
                csl = pl.ds(q * 16, 16)
                g0 = rows0[m, csl]
                g1 = rows1[m, csl]
                g2 = rows2[m, csl]
                sl = pl.ds(m * COUT + q * 16, 16)
                acc[sl] = acc[sl] + w0 * g0 + w1 * g1 + w2 * g2
            return 0

        lax.fori_loop(0, chunk, _mrow, 0)
        pltpu.sync_copy(
            acc, out_hbm.at[pl.ds((grow0 + c * chunk) * COUT, chunk * COUT)])


def _sc_call(p1f, p2f, h_rows, l_flat):
    mesh = plsc.VectorSubcoreMesh(core_axis_name="c", subcore_axis_name="s")
    fn = pl.kernel(
        _sc_body,
        out_type=jax.ShapeDtypeStruct((B * M * COUT,), jnp.float32),
        mesh=mesh,
        compiler_params=pltpu.CompilerParams(
            use_tc_tiling_on_sc=True, needs_layout_passes=False),
        scratch_types=[
            pltpu.VMEM((N,), jnp.float32),
            pltpu.VMEM((N,), jnp.float32),
            pltpu.VMEM((N,), jnp.float32),
            pltpu.VMEM((N,), jnp.float32),
            pltpu.VMEM((MPW,), jnp.float32),
            pltpu.VMEM((MPW,), jnp.float32),
            pltpu.VMEM((MPW,), jnp.float32),
            pltpu.VMEM((MPW,), jnp.float32),
            pltpu.VMEM((MPW,), jnp.float32),
            pltpu.VMEM((MPW,), jnp.float32),
            pltpu.VMEM((MPW,), jnp.int32),
            pltpu.VMEM((MPW,), jnp.int32),
            pltpu.VMEM((MPW,), jnp.int32),
            pltpu.VMEM((128, COUT), jnp.float32),
            pltpu.VMEM((128, COUT), jnp.float32),
            pltpu.VMEM((128, COUT), jnp.float32),
            pltpu.VMEM((128 * COUT,), jnp.float32),
            pltpu.SemaphoreType.DMA,
            pltpu.SemaphoreType.DMA,
            pltpu.SemaphoreType.DMA,
        ],
    )
    return fn(p1f[0], p1f[1], p1f[2], p2f[0], p2f[1], p2f[2], h_rows, l_flat)


@jax.jit
def kernel(x1, p1, x2, p2, W_up, gamma1, beta1, W_lat, gamma2, beta2):
    p1f = jnp.transpose(p1, (2, 0, 1)).reshape(3, B * N)
    p2f = jnp.transpose(p2, (2, 0, 1)).reshape(3, B * M)
    h = _h_call(x1, W_up.T, gamma1.reshape(1, COUT), beta1.reshape(1, COUT))
    l = _l_call(x2, W_lat.T, gamma2.reshape(1, COUT), beta2.reshape(1, COUT))
    out_flat = _sc_call(p1f, p2f, h.reshape(B * N, COUT),
                        l.reshape(B * M * COUT))
    out = jnp.transpose(out_flat.reshape(B, M, COUT), (0, 2, 1))
    return (out, p2)


# speedup vs baseline: 1.2135x; 1.1091x over previous
import functools

import jax
import jax.numpy as jnp
from jax import lax
from jax.experimental import pallas as pl
from jax.experimental.pallas import tpu as pltpu
from jax.experimental.pallas import tpu_sc as plsc

B, N, M, CIN, COUT = 4, 1024, 4096, 256, 128
CIN2 = COUT
NW = 32
MPW = B * M // NW
NG = MPW // 16
BN_EPS = 1e-5
_HIGH = jax.lax.Precision.HIGHEST


def _h_body(x1_ref, wt_ref, g_ref, b_ref, out_ref):
    wt = wt_ref[...]
    ys = [
        lax.dot_general(x1_ref[b], wt, (((0,), (0,)), ((), ())),
                        preferred_element_type=jnp.float32, precision=_HIGH)
        for b in range(B)
    ]
    tot = ys[0] + ys[1] + ys[2] + ys[3]
    mean = jnp.sum(tot, axis=0, keepdims=True) * (1.0 / (B * N))
    sq = sum(jnp.sum(y * y, axis=0, keepdims=True) for y in ys)
    var = sq * (1.0 / (B * N)) - mean * mean
    scale = g_ref[...] / jnp.sqrt(var + BN_EPS)
    shift = b_ref[...] - mean * scale
    for b in range(B):
        out_ref[b] = jnp.maximum(ys[b] * scale + shift, 0.0)


def _h_call(x1, wup_t, g1, b1):
    return pl.pallas_call(
        _h_body,
        out_shape=jax.ShapeDtypeStruct((B, N, COUT), jnp.float32),
    )(x1, wup_t, g1, b1)


def _l_body(x2_ref, wt_ref, g_ref, b_ref, out_ref, cov_ref, mu_ref):
    i = pl.program_id(0)

    @pl.when(i == 0)
    def _init():
        cov_ref[...] = jnp.zeros((CIN2, CIN2), jnp.float32)
        mu_ref[...] = jnp.zeros((CIN2, 1), jnp.float32)

    xb = x2_ref[0]

    @pl.when(i < B)
    def _accum():
        cov_ref[...] += lax.dot_general(
            xb, xb, (((1,), (1,)), ((), ())),
            preferred_element_type=jnp.float32, precision=_HIGH)
        mu_ref[...] += jnp.sum(xb, axis=1, keepdims=True)
        out_ref[...] = jnp.zeros_like(out_ref)

    @pl.when(i >= B)
    def _emit():
        wt = wt_ref[...]
        inv_l = 1.0 / (B * M)
        mean = lax.dot_general(mu_ref[...], wt, (((0,), (0,)), ((), ())),
                               preferred_element_type=jnp.float32,
                               precision=_HIGH) * inv_l
        t = lax.dot_general(cov_ref[...], wt, (((1,), (0,)), ((), ())),
                            preferred_element_type=jnp.float32,
                            precision=_HIGH)
        e2 = jnp.sum(t * wt, axis=0, keepdims=True) * inv_l
        var = e2 - mean * mean
        scale = g_ref[...] / jnp.sqrt(var + BN_EPS)
        shift = b_ref[...] - mean * scale
        wf = wt * scale
        y = lax.dot_general(xb, wf, (((0,), (0,)), ((), ())),
                            preferred_element_type=jnp.float32,
                            precision=_HIGH)
        out_ref[...] = jnp.maximum(y + shift, 0.0)[None]


def _l_call(x2, wlat_t, g2, b2):
    return pl.pallas_call(
        _l_body,
        grid=(2 * B,),
        in_specs=[
            pl.BlockSpec((1, CIN2, M), lambda i: (lax.rem(i, B), 0, 0)),
            pl.BlockSpec((CIN2, COUT), lambda i: (0, 0)),
            pl.BlockSpec((1, COUT), lambda i: (0, 0)),
            pl.BlockSpec((1, COUT), lambda i: (0, 0)),
        ],
        out_specs=pl.BlockSpec(
            (1, M, COUT), lambda i: (jnp.maximum(i - B, 0), 0, 0)),
        out_shape=jax.ShapeDtypeStruct((B, M, COUT), jnp.float32),
        scratch_shapes=[
            pltpu.VMEM((CIN2, CIN2), jnp.float32),
            pltpu.VMEM((CIN2, 1), jnp.float32),
        ],
    )(x2, wlat_t, g2, b2)


def _bf16_round(x):
    u = plsc.bitcast(x, jnp.uint32)
    r = (u + jnp.uint32(0x7FFF) + ((u >> jnp.uint32(16)) & jnp.uint32(1)))
    r = r & jnp.uint32(0xFFFF0000)
    return plsc.bitcast(r, jnp.float32)


def _sc_nn_body(p1x, p1y, p1z, p2x, p2y, p2z,
                i0h, i1h, i2h, w0h, w1h, w2h,
                cx, cy, cz, cn, fxr, fyr, fzr,
                wb0, wb1, wb2, ib0, ib1, ib2):
    nc = 2
    wid = lax.axis_index("s") * nc + lax.axis_index("c")
    b = wid // (NW // B)
    mbase = b * M + (wid % (NW // B)) * MPW

    pltpu.sync_copy(p1x.at[pl.ds(b * N, N)], cx)
    pltpu.sync_copy(p1y.at[pl.ds(b * N, N)], cy)
    pltpu.sync_copy(p1z.at[pl.ds(b * N, N)], cz)
    pltpu.sync_copy(p2x.at[pl.ds(mbase, MPW)], fxr)
    pltpu.sync_copy(p2y.at[pl.ds(mbase, MPW)], fyr)
    pltpu.sync_copy(p2z.at[pl.ds(mbase, MPW)], fzr)

    def _cn_body(i, _):
        sl = pl.ds(i * 16, 16)
        vx = cx[sl]
        vy = cy[sl]
        vz = cz[sl]
        cn[sl] = vx * vx + vy * vy + vz * vz
        cx[sl] = _bf16_round(vx) * 2.0
        cy[sl] = _bf16_round(vy) * 2.0
        cz[sl] = _bf16_round(vz) * 2.0
        return 0

    lax.fori_loop(0, N // 16, _cn_body, 0)

    kbig = jnp.full((16,), 0xFFFFFFFF, jnp.uint32)
    hi = jnp.full((16,), 0x80000000, jnp.uint32)
    msk = jnp.full((16,), 0xFFFFFC00, jnp.uint32)
    idxm = jnp.full((16,), 0x3FF, jnp.uint32)

    def _group(gp, _):
        sls = [pl.ds(gp * 64 + 16 * t, 16) for t in range(4)]
        fxs, fys, fzs, fns = [], [], [], []
        for sl in sls:
            fx = fxr[sl]
            fy = fyr[sl]
            fz = fzr[sl]
            fns.append(fx * fx + fy * fy + fz * fz)
            fxs.append(_bf16_round(fx))
            fys.append(_bf16_round(fy))
            fzs.append(_bf16_round(fz))

        def _scan(n, carry):
            nv = jnp.broadcast_to(n, (16,)).astype(jnp.int32)
            nvu = plsc.bitcast(nv, jnp.uint32)
            cxv = plsc.load_gather(cx, [nv])
            cyv = plsc.load_gather(cy, [nv])
            czv = plsc.load_gather(cz, [nv])
            cnv = plsc.load_gather(cn, [nv])
            out = []
            for t in range(4):
                k1, k2, k3 = carry[3 * t:3 * t + 3]
                dot2 = fxs[t] * cxv + fys[t] * cyv + fzs[t] * czv
                d = (fns[t] + cnv) - dot2
                di = plsc.bitcast(d, jnp.int32)
                sgn = plsc.bitcast(di >> 31, jnp.uint32)
                key = plsc.bitcast(di, jnp.uint32) ^ (sgn | hi)
                key = (key & msk) | nvu
                a = jnp.minimum(key, k1)
                bq = jnp.maximum(key, k1)
                c = jnp.minimum(bq, k2)
                e = jnp.maximum(bq, k2)
                out += [a, c, jnp.minimum(e, k3)]
            return tuple(out)

        res = lax.fori_loop(0, N, _scan, (kbig,) * 12, unroll=2)

        base = jnp.broadcast_to(b * N, (16,)).astype(jnp.int32)
        for t, sl in enumerate(sls):
            rs = []
            for j in range(3):
                kk = res[3 * t + j]
                ik = plsc.bitcast(kk & idxm, jnp.int32)
                cxv = plsc.load_gather(cx, [ik])
                cyv = plsc.load_gather(cy, [ik])
                czv = plsc.load_gather(cz, [ik])
                cnv = plsc.load_gather(cn, [ik])
                dk = (fns[t] + cnv) - (fxs[t] * cxv + fys[t] * cyv
                                       + fzs[t] * czv)
                rs.append(1.0 / (dk + 1e-8))
                [ib0, ib1, ib2][j][sl] = ik + base
            s = 1.0 / (rs[0] + rs[1] + rs[2])
            wb0[sl] = rs[0] * s
            wb1[sl] = rs[1] * s
            wb2[sl] = rs[2] * s
        return 0

    lax.fori_loop(0, NG // 4, _group, 0)

    wsl = pl.ds(wid * MPW, MPW)
    pltpu.sync_copy(ib0, i0h.at[wsl])
    pltpu.sync_copy(ib1, i1h.at[wsl])
    pltpu.sync_copy(ib2, i2h.at[wsl])
    pltpu.sync_copy(wb0, w0h.at[wsl])
    pltpu.sync_copy(wb1, w1h.at[wsl])
    pltpu.sync_copy(wb2, w2h.at[wsl])


def _sc_nn_call(p1f, p2f):
    mesh = plsc.VectorSubcoreMesh(core_axis_name="c", subcore_axis_name="s")
    fn = pl.kernel(
        _sc_nn_body,
        out_type=tuple([jax.ShapeDtypeStruct((B * M,), jnp.int32)] * 3
                       + [jax.ShapeDtypeStruct((B * M,), jnp.float32)] * 3),
        mesh=mesh,
        compiler_params=pltpu.CompilerParams(
            use_tc_tiling_on_sc=True, needs_layout_passes=False),
        scratch_types=[pltpu.VMEM((N,), jnp.float32)] * 4
                      + [pltpu.VMEM((MPW,), jnp.float32)] * 3
                      + [pltpu.VMEM((MPW,), jnp.float32)] * 3
                      + [pltpu.VMEM((MPW,), jnp.int32)] * 3,
    )
    return fn(p1f[0], p1f[1], p1f[2], p2f[0], p2f[1], p2f[2])


def _sc_interp_body(h_hbm, l_hbm, i0h, i1h, i2h, w0h, w1h, w2h, out_hbm,
                    wb0, wb1, wb2, ib0, ib1, ib2,
                    rows0, rows1, rows2, acc, sem0, sem1, sem2):
    nc = 2
    wid = lax.axis_index("s") * nc + lax.axis_index("c")
    grow0 = wid * MPW
    wsl = pl.ds(grow0, MPW)
    pltpu.sync_copy(i0h.at[wsl], ib0)
    pltpu.sync_copy(i1h.at[wsl], ib1)
    pltpu.sync_copy(i2h.at[wsl], ib2)
    pltpu.sync_copy(w0h.at[wsl], wb0)
    pltpu.sync_copy(w1h.at[wsl], wb1)
    pltpu.sync_copy(w2h.at[wsl], wb2)

    rows = [rows0, rows1, rows2]
    sems = [sem0, sem1, sem2]
    ibufs = [ib0, ib1, ib2]
    chunk = 128
    for c in range(MPW // chunk):
        cps = []
        for k in range(3):
            cp = pltpu.make_async_copy(
                h_hbm.at[ibufs[k].at[pl.ds(c * chunk, chunk)]], rows[k],
                sems[k])
            cp.start()
            cps.append(cp)
        pltpu.sync_copy(
            l_hbm.at[pl.ds((grow0 + c * chunk) * COUT, chunk * COUT)], acc)
        for cp in cps:
            cp.wait()

        def _mrow(m, _):
            mv = jnp.broadcast_to(c * chunk + m, (16,)).astype(jnp.int32)
            w0 = plsc.load_gather(wb0, [mv])
            w1 = plsc.load_gather(wb1, [mv])
            w2 = plsc.load_gather(wb2, [mv])
            for q in range(8):
                csl = pl.ds(q * 16, 16)
                g0 = rows0[m, csl]
                g1 = rows1[m, csl]
                g2 = rows2[m, csl]
                sl = pl.ds(m * COUT + q * 16, 16)
                acc[sl] = acc[sl] + w0 * g0 + w1 * g1 + w2 * g2
            return 0

        lax.fori_loop(0, chunk, _mrow, 0)
        pltpu.sync_copy(
            acc, out_hbm.at[pl.ds((grow0 + c * chunk) * COUT, chunk * COUT)])


def _sc_interp_call(h_rows, l_flat, i0, i1, i2, w0, w1, w2):
    mesh = plsc.VectorSubcoreMesh(core_axis_name="c", subcore_axis_name="s")
    fn = pl.kernel(
        _sc_interp_body,
        out_type=jax.ShapeDtypeStruct((B * M * COUT,), jnp.float32),
        mesh=mesh,
        compiler_params=pltpu.CompilerParams(
            use_tc_tiling_on_sc=True, needs_layout_passes=False),
        scratch_types=[pltpu.VMEM((MPW,), jnp.float32)] * 3
                      + [pltpu.VMEM((MPW,), jnp.int32)] * 3
                      + [pltpu.VMEM((128, COUT), jnp.float32)] * 3
                      + [pltpu.VMEM((128 * COUT,), jnp.float32),
                         pltpu.SemaphoreType.DMA,
                         pltpu.SemaphoreType.DMA,
                         pltpu.SemaphoreType.DMA],
    )
    return fn(h_rows, l_flat, i0, i1, i2, w0, w1, w2)


@jax.jit
def kernel(x1, p1, x2, p2, W_up, gamma1, beta1, W_lat, gamma2, beta2):
    p1f = jnp.transpose(p1, (2, 0, 1)).reshape(3, B * N)
    p2f = jnp.transpose(p2, (2, 0, 1)).reshape(3, B * M)
    nn = _sc_nn_call(p1f, p2f)
    h = _h_call(x1, W_up.T, gamma1.reshape(1, COUT), beta1.reshape(1, COUT))
    l = _l_call(x2, W_lat.T, gamma2.reshape(1, COUT), beta2.reshape(1, COUT))
    out_flat = _sc_interp_call(h.reshape(B * N, COUT),
                               l.reshape(B * M * COUT), *nn)
    out = jnp.transpose(out_flat.reshape(B, M, COUT), (0, 2, 1))
    return (out, p2)
